# baseline (speedup 1.0000x reference)
import functools

import jax
import jax.numpy as jnp
import numpy as np
from jax import lax
from jax.experimental import pallas as pl
from jax.experimental.pallas import tpu as pltpu
from jax.experimental.pallas import tpu_sc as plsc

_B = 4096
_H = 200
_D = 32
_DP = _D // 2
_NW = 32
_BPW = _B // _NW
_C0 = 128
_C1 = _H - _C0
_NBUF = 4


def _pack_body(e_ref, o_ref):
    v = jax.lax.bitcast_convert_type(e_ref[...], jnp.int32)
    r = jax.lax.shift_right_logical(
        v + 0x7FFF + (jax.lax.shift_right_logical(v, 16) & 1), 16)
    first = r[:, 0:_DP]
    second = r[:, _DP:_D]
    o_ref[...] = first | jax.lax.shift_left(second, 16)


_PACK_ROWS = 8000


def _pack_table(emb):
    v = emb.shape[0]
    return pl.pallas_call(
        _pack_body,
        grid=(v // _PACK_ROWS,),
        in_specs=[pl.BlockSpec((_PACK_ROWS, _D), lambda i: (i, 0))],
        out_specs=pl.BlockSpec((_PACK_ROWS, _DP), lambda i: (i, 0)),
        out_shape=jax.ShapeDtypeStruct((v, _DP), jnp.int32),
    )(emb)


def _pool_body(x_hbm, emb_hbm, out_hbm, idx_v, rows_v, acc_v, *sems):
    c = lax.axis_index("c")
    s = lax.axis_index("s")
    wid = s * 2 + c
    base = wid * _BPW
    pltpu.sync_copy(x_hbm.at[pl.ds(base, _BPW)], idx_v)

    def issue(b, buf):
        pltpu.async_copy(emb_hbm.at[idx_v.at[b, pl.ds(0, _C0)]],
                         rows_v.at[buf, pl.ds(0, _C0)], sems[buf])
        pltpu.async_copy(emb_hbm.at[idx_v.at[b, pl.ds(_C0, _C1)]],
                         rows_v.at[buf, pl.ds(_C0, _C1)], sems[buf])

    def drain(b, buf):
        pltpu.make_async_copy(emb_hbm.at[idx_v.at[b, pl.ds(0, _C0)]],
                              rows_v.at[buf, pl.ds(0, _C0)], sems[buf]).wait()
        pltpu.make_async_copy(emb_hbm.at[idx_v.at[b, pl.ds(_C0, _C1)]],
                              rows_v.at[buf, pl.ds(_C0, _C1)], sems[buf]).wait()

    for p in range(_NBUF):
        issue(p, p)

    hi_mask = jnp.full((16,), -65536, dtype=jnp.int32)

    def one_group(t, carry):
        for p in range(_NBUF):
            b = _NBUF * t + p
            drain(b, p)

            def red(i, acc):
                new = list(acc)
                for k in range(2):
                    v = rows_v[p, 2 * i + k, 0:16]
                    lo = plsc.bitcast(jax.lax.shift_left(v, 16), jnp.float32)
                    hi = plsc.bitcast(v & hi_mask, jnp.float32)
                    new[2 * k] = new[2 * k] + lo
                    new[2 * k + 1] = new[2 * k + 1] + hi
                return tuple(new)

            z = jnp.zeros((16,), jnp.float32)
            acc = lax.fori_loop(0, _H // 2, red, (z,) * 4, unroll=8)
            acc_v[b, 0:16] = acc[0] + acc[2]
            acc_v[b, 16:32] = acc[1] + acc[3]

            @pl.when(b + _NBUF < _BPW)
            def _():
                issue(b + _NBUF, p)
        return carry

    lax.fori_loop(0, _BPW // _NBUF, one_group, 0)
    pltpu.sync_copy(acc_v, out_hbm.at[pl.ds(base, _BPW)])


_pool = functools.partial(
    pl.kernel,
    out_type=jax.ShapeDtypeStruct((_B, _D), jnp.float32),
    mesh=plsc.VectorSubcoreMesh(core_axis_name="c", subcore_axis_name="s"),
    scratch_types=[
        pltpu.VMEM((_BPW, _H), jnp.int32),
        pltpu.VMEM((_NBUF, _H, _DP), jnp.int32),
        pltpu.VMEM((_BPW, _D), jnp.float32),
    ] + [pltpu.SemaphoreType.DMA] * _NBUF,
    compiler_params=pltpu.CompilerParams(use_tc_tiling_on_sc=False,
                                         needs_layout_passes=False),
)(_pool_body)

def _mlp_body(p_ref, w1_ref, b1_ref, w2_ref, b2_ref, o_ref):
    h = p_ref[...] * (1.0 / _H)
    h = jnp.dot(h, w1_ref[...], preferred_element_type=jnp.float32) + b1_ref[...]
    h = jnp.maximum(h, 0.0)
    o_ref[...] = jnp.dot(h, w2_ref[...], preferred_element_type=jnp.float32) + b2_ref[...]


def kernel(x, emb, W1, b1, W2, b2):
    packed = _pack_table(emb)
    pooled = _pool(x, packed)
    w1p = W1
    w2p = jnp.zeros((_D, 128), jnp.float32).at[:, :10].set(W2)
    b2p = jnp.zeros((1, 128), jnp.float32).at[:, :10].set(b2)
    out = pl.pallas_call(
        _mlp_body,
        out_shape=jax.ShapeDtypeStruct((_B, 128), jnp.float32),
    )(pooled, w1p, b1.reshape(1, _D), w2p, b2p)
    return out[:, :10]


# speedup vs baseline: 1.1326x; 1.1326x over previous
import functools

import jax
import jax.numpy as jnp
import numpy as np
from jax import lax
from jax.experimental import pallas as pl
from jax.experimental.pallas import tpu as pltpu
from jax.experimental.pallas import tpu_sc as plsc

_B = 4096
_H = 200
_D = 32
_NW = 32
_BPW = _B // _NW
_C0 = 128
_C1 = _H - _C0
_NBUF = 4


def _conv_body(e_ref, o_ref):
    o_ref[...] = e_ref[...].astype(jnp.bfloat16)


_CONV_ROWS = 2000


def _conv_table(emb):
    wide = emb.reshape(-1, 128)
    v = wide.shape[0]
    out = pl.pallas_call(
        _conv_body,
        grid=(v // _CONV_ROWS,),
        in_specs=[pl.BlockSpec((_CONV_ROWS, 128), lambda i: (i, 0))],
        out_specs=pl.BlockSpec((_CONV_ROWS, 128), lambda i: (i, 0)),
        out_shape=jax.ShapeDtypeStruct((v, 128), jnp.bfloat16),
    )(wide)
    return out.reshape(emb.shape[0], _D)


def _pool_body(x_hbm, emb_hbm, out_hbm, idx_v, rows_v, acc_v, *sems):
    c = lax.axis_index("c")
    s = lax.axis_index("s")
    wid = s * 2 + c
    base = wid * _BPW
    pltpu.sync_copy(x_hbm.at[pl.ds(base, _BPW)], idx_v)

    def issue(b, buf):
        pltpu.async_copy(emb_hbm.at[idx_v.at[b, pl.ds(0, _C0)]],
                         rows_v.at[buf, pl.ds(0, _C0)], sems[buf])
        pltpu.async_copy(emb_hbm.at[idx_v.at[b, pl.ds(_C0, _C1)]],
                         rows_v.at[buf, pl.ds(_C0, _C1)], sems[buf])

    def drain(b, buf):
        pltpu.make_async_copy(emb_hbm.at[idx_v.at[b, pl.ds(0, _C0)]],
                              rows_v.at[buf, pl.ds(0, _C0)], sems[buf]).wait()
        pltpu.make_async_copy(emb_hbm.at[idx_v.at[b, pl.ds(_C0, _C1)]],
                              rows_v.at[buf, pl.ds(_C0, _C1)], sems[buf]).wait()

    for p in range(_NBUF):
        issue(p, p)

    hi_mask = jnp.full((16,), -65536, dtype=jnp.int32)

    def one_group(t, carry):
        for p in range(_NBUF):
            b = _NBUF * t + p
            drain(b, p)

            def red(i, acc):
                new = list(acc)
                for k in range(2):
                    j = 2 * i + k
                    v = plsc.bitcast(rows_v[p, j, 0:32], jnp.int32)
                    lo = plsc.bitcast(jax.lax.shift_left(v, 16), jnp.float32)
                    hi = plsc.bitcast(v & hi_mask, jnp.float32)
                    new[2 * k] = new[2 * k] + lo
                    new[2 * k + 1] = new[2 * k + 1] + hi
                return tuple(new)

            z = jnp.zeros((16,), jnp.float32)
            acc = lax.fori_loop(0, _H // 2, red, (z,) * 4, unroll=8)
            acc_v[b, 0:16] = acc[0] + acc[2]
            acc_v[b, 16:32] = acc[1] + acc[3]

            @pl.when(b + _NBUF < _BPW)
            def _():
                issue(b + _NBUF, p)
        return carry

    lax.fori_loop(0, _BPW // _NBUF, one_group, 0)
    pltpu.sync_copy(acc_v, out_hbm.at[pl.ds(base, _BPW)])


_pool = functools.partial(
    pl.kernel,
    out_type=jax.ShapeDtypeStruct((_B, _D), jnp.float32),
    mesh=plsc.VectorSubcoreMesh(core_axis_name="c", subcore_axis_name="s"),
    scratch_types=[
        pltpu.VMEM((_BPW, _H), jnp.int32),
        pltpu.VMEM((_NBUF, _H, _D), jnp.bfloat16),
        pltpu.VMEM((_BPW, _D), jnp.float32),
    ] + [pltpu.SemaphoreType.DMA] * _NBUF,
    compiler_params=pltpu.CompilerParams(use_tc_tiling_on_sc=False,
                                         needs_layout_passes=False),
)(_pool_body)

_PERM = np.array(list(range(0, _D, 2)) + list(range(1, _D, 2)))


def _mlp_body(p_ref, w1_ref, b1_ref, w2_ref, b2_ref, o_ref):
    h = p_ref[...] * (1.0 / _H)
    h = jnp.dot(h, w1_ref[...], preferred_element_type=jnp.float32) + b1_ref[...]
    h = jnp.maximum(h, 0.0)
    o_ref[...] = jnp.dot(h, w2_ref[...], preferred_element_type=jnp.float32) + b2_ref[...]


def kernel(x, emb, W1, b1, W2, b2):
    emb_bf = _conv_table(emb)
    pooled = _pool(x, emb_bf)
    w1p = W1[_PERM, :]
    w2p = jnp.zeros((_D, 128), jnp.float32).at[:, :10].set(W2)
    b2p = jnp.zeros((1, 128), jnp.float32).at[:, :10].set(b2)
    out = pl.pallas_call(
        _mlp_body,
        out_shape=jax.ShapeDtypeStruct((_B, 128), jnp.float32),
    )(pooled, w1p, b1.reshape(1, _D), w2p, b2p)
    return out[:, :10]


# speedup vs baseline: 1.3147x; 1.1608x over previous
import functools

import jax
import jax.numpy as jnp
from jax import lax
from jax.experimental import pallas as pl
from jax.experimental.pallas import tpu as pltpu
from jax.experimental.pallas import tpu_sc as plsc

_B = 4096
_H = 200
_D = 32
_DP = _D // 2
_V = 1000000
_NW = 32
_BPW = _B // _NW
_C0 = 128
_C1 = _H - _C0
_NBUF = 4

_RPW = _V // _NW
_PCH = 625
_PITERS = _RPW // _PCH


def _pack_body(emb_hbm, out_hbm, in_v, out_v, *sems):
    c = lax.axis_index("c")
    s = lax.axis_index("s")
    wid = s * 2 + c
    base = wid * _RPW

    def issue(t, buf):
        pltpu.async_copy(emb_hbm.at[pl.ds(2 * (base + t * _PCH), 2 * _PCH)],
                         in_v.at[buf], sems[buf])

    def drain(t, buf):
        pltpu.make_async_copy(
            emb_hbm.at[pl.ds(2 * (base + t * _PCH), 2 * _PCH)],
            in_v.at[buf], sems[buf]).wait()

    for p in range(2):
        issue(p, p)

    hi_mask = jnp.full((16,), -65536, dtype=jnp.int32)
    half = jnp.full((16,), 0x8000, dtype=jnp.int32)

    def one_group(t2, carry):
        for p in range(2):
            t = 2 * t2 + p
            drain(t, p)

            def pk(i, carry2):
                a = plsc.bitcast(in_v[p, 2 * i, 0:16], jnp.int32)
                b = plsc.bitcast(in_v[p, 2 * i + 1, 0:16], jnp.int32)
                lo = jax.lax.shift_right_logical(a + half, 16)
                hi = (b + half) & hi_mask
                out_v[p, i, 0:16] = hi | lo
                return carry2

            lax.fori_loop(0, _PCH, pk, 0, unroll=8)
            pltpu.sync_copy(out_v.at[p], out_hbm.at[pl.ds(base + t * _PCH, _PCH)])

            @pl.when(t + 2 < _PITERS)
            def _():
                issue(t + 2, p)
        return carry

    lax.fori_loop(0, _PITERS // 2, one_group, 0)


_pack = functools.partial(
    pl.kernel,
    out_type=jax.ShapeDtypeStruct((_V, _DP), jnp.int32),
    mesh=plsc.VectorSubcoreMesh(core_axis_name="c", subcore_axis_name="s"),
    scratch_types=[
        pltpu.VMEM((2, 2 * _PCH, _DP), jnp.float32),
        pltpu.VMEM((2, _PCH, _DP), jnp.int32),
        pltpu.SemaphoreType.DMA,
        pltpu.SemaphoreType.DMA,
    ],
    compiler_params=pltpu.CompilerParams(use_tc_tiling_on_sc=False,
                                         needs_layout_passes=False),
)(_pack_body)


def _pool_body(x_hbm, emb_hbm, out_hbm, idx_v, rows_v, acc_v, *sems):
    c = lax.axis_index("c")
    s = lax.axis_index("s")
    wid = s * 2 + c
    base = wid * _BPW
    pltpu.sync_copy(x_hbm.at[pl.ds(base, _BPW)], idx_v)

    def issue(b, buf):
        pltpu.async_copy(emb_hbm.at[idx_v.at[b, pl.ds(0, _C0)]],
                         rows_v.at[buf, pl.ds(0, _C0)], sems[buf])
        pltpu.async_copy(emb_hbm.at[idx_v.at[b, pl.ds(_C0, _C1)]],
                         rows_v.at[buf, pl.ds(_C0, _C1)], sems[buf])

    def drain(b, buf):
        pltpu.make_async_copy(emb_hbm.at[idx_v.at[b, pl.ds(0, _C0)]],
                              rows_v.at[buf, pl.ds(0, _C0)], sems[buf]).wait()
        pltpu.make_async_copy(emb_hbm.at[idx_v.at[b, pl.ds(_C0, _C1)]],
                              rows_v.at[buf, pl.ds(_C0, _C1)], sems[buf]).wait()

    for p in range(_NBUF):
        issue(p, p)

    hi_mask = jnp.full((16,), -65536, dtype=jnp.int32)

    def one_group(t, carry):
        for p in range(_NBUF):
            b = _NBUF * t + p
            drain(b, p)

            def red(i, acc):
                new = list(acc)
                for k in range(2):
                    v = rows_v[p, 2 * i + k, 0:16]
                    lo = plsc.bitcast(jax.lax.shift_left(v, 16), jnp.float32)
                    hi = plsc.bitcast(v & hi_mask, jnp.float32)
                    new[2 * k] = new[2 * k] + lo
                    new[2 * k + 1] = new[2 * k + 1] + hi
                return tuple(new)

            z = jnp.zeros((16,), jnp.float32)
            acc = lax.fori_loop(0, _H // 2, red, (z,) * 4, unroll=8)
            acc_v[b, 0:16] = acc[0] + acc[2]
            acc_v[b, 16:32] = acc[1] + acc[3]

            @pl.when(b + _NBUF < _BPW)
            def _():
                issue(b + _NBUF, p)
        return carry

    lax.fori_loop(0, _BPW // _NBUF, one_group, 0)
    pltpu.sync_copy(acc_v, out_hbm.at[pl.ds(base, _BPW)])


_pool = functools.partial(
    pl.kernel,
    out_type=jax.ShapeDtypeStruct((_B, _D), jnp.float32),
    mesh=plsc.VectorSubcoreMesh(core_axis_name="c", subcore_axis_name="s"),
    scratch_types=[
        pltpu.VMEM((_BPW, _H), jnp.int32),
        pltpu.VMEM((_NBUF, _H, _DP), jnp.int32),
        pltpu.VMEM((_BPW, _D), jnp.float32),
    ] + [pltpu.SemaphoreType.DMA] * _NBUF,
    compiler_params=pltpu.CompilerParams(use_tc_tiling_on_sc=False,
                                         needs_layout_passes=False),
)(_pool_body)


def _mlp_body(p_ref, w1_ref, b1_ref, w2_ref, b2_ref, o_ref):
    h = p_ref[...] * (1.0 / _H)
    h = jnp.dot(h, w1_ref[...], preferred_element_type=jnp.float32) + b1_ref[...]
    h = jnp.maximum(h, 0.0)
    o_ref[...] = jnp.dot(h, w2_ref[...], preferred_element_type=jnp.float32) + b2_ref[...]


def kernel(x, emb, W1, b1, W2, b2):
    packed = _pack(emb.reshape(2 * _V, _DP))
    pooled = _pool(x, packed)
    w2p = jnp.zeros((_D, 128), jnp.float32).at[:, :10].set(W2)
    b2p = jnp.zeros((1, 128), jnp.float32).at[:, :10].set(b2)
    out = pl.pallas_call(
        _mlp_body,
        out_shape=jax.ShapeDtypeStruct((_B, 128), jnp.float32),
    )(pooled, W1, b1.reshape(1, _D), w2p, b2p)
    return out[:, :10]


# speedup vs baseline: 1.7863x; 1.3587x over previous
import functools

import jax
import jax.numpy as jnp
from jax import lax
from jax.experimental import pallas as pl
from jax.experimental.pallas import tpu as pltpu
from jax.experimental.pallas import tpu_sc as plsc

_B = 4096
_H = 200
_D = 32
_NW = 32
_BPW = _B // _NW
_C0 = 128
_C1 = _H - _C0
_NBUF = 4


def _pool_body(x_hbm, emb_hbm, out_hbm, idx_v, rows_v, acc_v, *sems):
    c = lax.axis_index("c")
    s = lax.axis_index("s")
    wid = s * 2 + c
    base = wid * _BPW
    pltpu.sync_copy(x_hbm.at[pl.ds(base, _BPW)], idx_v)

    def issue(b, buf):
        pltpu.async_copy(emb_hbm.at[idx_v.at[b, pl.ds(0, _C0)]],
                         rows_v.at[buf, pl.ds(0, _C0)], sems[buf])
        pltpu.async_copy(emb_hbm.at[idx_v.at[b, pl.ds(_C0, _C1)]],
                         rows_v.at[buf, pl.ds(_C0, _C1)], sems[buf])

    def drain(b, buf):
        pltpu.make_async_copy(emb_hbm.at[idx_v.at[b, pl.ds(0, _C0)]],
                              rows_v.at[buf, pl.ds(0, _C0)], sems[buf]).wait()
        pltpu.make_async_copy(emb_hbm.at[idx_v.at[b, pl.ds(_C0, _C1)]],
                              rows_v.at[buf, pl.ds(_C0, _C1)], sems[buf]).wait()

    for p in range(_NBUF):
        issue(p, p)

    def one_group(t, carry):
        for p in range(_NBUF):
            b = _NBUF * t + p
            drain(b, p)

            def red(i, acc):
                new = list(acc)
                for k in range(2):
                    j = 2 * i + k
                    new[2 * k] = new[2 * k] + rows_v[p, j, 0:16]
                    new[2 * k + 1] = new[2 * k + 1] + rows_v[p, j, 16:32]
                return tuple(new)

            z = jnp.zeros((16,), jnp.float32)
            acc = lax.fori_loop(0, _H // 2, red, (z,) * 4, unroll=8)
            acc_v[b, 0:16] = acc[0] + acc[2]
            acc_v[b, 16:32] = acc[1] + acc[3]

            @pl.when(b + _NBUF < _BPW)
            def _():
                issue(b + _NBUF, p)
        return carry

    lax.fori_loop(0, _BPW // _NBUF, one_group, 0)
    pltpu.sync_copy(acc_v, out_hbm.at[pl.ds(base, _BPW)])


_pool = functools.partial(
    pl.kernel,
    out_type=jax.ShapeDtypeStruct((_B, _D), jnp.float32),
    mesh=plsc.VectorSubcoreMesh(core_axis_name="c", subcore_axis_name="s"),
    scratch_types=[
        pltpu.VMEM((_BPW, _H), jnp.int32),
        pltpu.VMEM((_NBUF, _H, _D), jnp.float32),
        pltpu.VMEM((_BPW, _D), jnp.float32),
    ] + [pltpu.SemaphoreType.DMA] * _NBUF,
    compiler_params=pltpu.CompilerParams(use_tc_tiling_on_sc=False,
                                         needs_layout_passes=False),
)(_pool_body)


def _mlp_body(p_ref, w1_ref, b1_ref, w2_ref, b2_ref, o_ref):
    h = p_ref[...] * (1.0 / _H)
    h = jnp.dot(h, w1_ref[...], preferred_element_type=jnp.float32) + b1_ref[...]
    h = jnp.maximum(h, 0.0)
    o_ref[...] = jnp.dot(h, w2_ref[...], preferred_element_type=jnp.float32) + b2_ref[...]


def kernel(x, emb, W1, b1, W2, b2):
    pooled = _pool(x, emb)
    w2p = jnp.zeros((_D, 128), jnp.float32).at[:, :10].set(W2)
    b2p = jnp.zeros((1, 128), jnp.float32).at[:, :10].set(b2)
    out = pl.pallas_call(
        _mlp_body,
        out_shape=jax.ShapeDtypeStruct((_B, 128), jnp.float32),
    )(pooled, W1, b1.reshape(1, _D), w2p, b2p)
    return out[:, :10]
